# Initial kernel scaffold; baseline (speedup 1.0000x reference)
import jax
import jax.numpy as jnp
from jax.experimental import pallas as pl


def kernel(vectors, node_species, senders, receivers, graph_id, species_table, W1_0, W2_0, Wsc_0, Wspec_0, lns_0, W1_1, W2_1, Wsc_1, Wspec_1, lns_1, W1_2, W2_2, Wsc_2, Wspec_2, lns_2, W_ro, hW1, hW2):
    raise NotImplementedError("write your pallas kernel here")


# speedup vs baseline: 1.9874x; 1.9874x over previous
import functools

_INTERP = False

import jax
import jax.numpy as jnp
from jax import lax
from jax.experimental import pallas as pl
from jax.experimental.pallas import tpu as pltpu
from jax.experimental.pallas import tpu_sc as plsc

_N = 10000
_E = 320000
_D = 128
_NB = 8
_HB = 32
_NSH = 9
_NSP = 89
_G = 64
_RMAX = 5.0
_AVG = 32.0

_BE = 4000
_BN = 400
_NBLK = _N // _BN

_NC = 2
_NS = 16
_NW = _NC * _NS
_EPW = _E // _NW
_K = 80
_NCH = _EPW // _K
_RCH = 80
_NRCH = _N // _RCH


def _silu(t):
    return t * (1.0 / (1.0 + jnp.exp(-t)))


def _edge_coeff_body(vec_ref, w10, w20, w11, w21, w12, w22,
                     c0_ref, c1_ref, c2_ref):
    v = -vec_ref[...]
    x2 = jnp.sum(v * v, axis=-1, keepdims=True)
    r = jnp.where(x2 == 0.0, 0.0,
                  jnp.sqrt(jnp.where(x2 == 0.0, 1.0, x2)))
    inv = 1.0 / jnp.where(r == 0.0, 1.0, r)
    n = v * inv
    nx, ny, nz = n[:, 0:1], n[:, 1:2], n[:, 2:3]
    s3 = 3.0 ** 0.5
    s15 = 15.0 ** 0.5
    sh = jnp.concatenate([
        jnp.ones_like(nx),
        s3 * nx, s3 * ny, s3 * nz,
        s15 * nx * ny,
        s15 * ny * nz,
        ((5.0 ** 0.5) / 2.0) * (3.0 * nz * nz - 1.0),
        s15 * nx * nz,
        (s15 / 2.0) * (nx * nx - ny * ny),
    ], axis=1)

    rs = jnp.where(r == 0.0, 1.0, r)
    nvals = lax.broadcasted_iota(jnp.int32, (1, _NB), 1).astype(jnp.float32) + 1.0
    basis = ((2.0 / _RMAX) ** 0.5) * jnp.sin(nvals * (jnp.pi / _RMAX) * r) / rs
    u = jnp.clip(r * (1.0 / _RMAX), 0.0, 1.0)
    u3 = u * u * u
    env = 1.0 - 10.0 * u3 + 15.0 * u3 * u - 6.0 * u3 * u * u
    rad = basis * env

    for wa, wb, c_ref in ((w10, w20, c0_ref), (w11, w21, c1_ref),
                          (w12, w22, c2_ref)):
        h = _silu(jnp.dot(rad, wa[...], preferred_element_type=jnp.float32))
        w = jnp.dot(h, wb[...], preferred_element_type=jnp.float32)
        c = jnp.sum(w * sh, axis=-1, keepdims=True)
        c_ref[...] = c * (1.0 / _AVG)


def _edge_coeffs(vectors, W1_0, W2_0, W1_1, W2_1, W1_2, W2_2):
    wspec = pl.BlockSpec((_NB, _HB), lambda i: (0, 0))
    w2spec = pl.BlockSpec((_HB, _NSH), lambda i: (0, 0))
    cspec = pl.BlockSpec((_BE, 1), lambda i: (i, 0))
    return pl.pallas_call(
        _edge_coeff_body,
        grid=(_E // _BE,),
        in_specs=[pl.BlockSpec((_BE, 3), lambda i: (i, 0)),
                  wspec, w2spec, wspec, w2spec, wspec, w2spec],
        out_specs=[cspec, cspec, cspec],
        out_shape=[jax.ShapeDtypeStruct((_E, 1), jnp.float32)] * 3,
        interpret=_INTERP,
    )(vectors, W1_0, W2_0, W1_1, W2_1, W1_2, W2_2)


def _embed_body(sp_ref, table_ref, out_ref):
    s = sp_ref[0, 0, :]
    oh = (s[:, None] == lax.broadcasted_iota(jnp.int32, (_BN, _NSP), 1)
          ).astype(jnp.float32)
    out_ref[...] = jnp.dot(oh, table_ref[...],
                           preferred_element_type=jnp.float32)


def _embed(species3, table):
    return pl.pallas_call(
        _embed_body,
        grid=(_NBLK,),
        in_specs=[pl.BlockSpec((1, 1, _BN), lambda i: (i, 0, 0)),
                  pl.BlockSpec((_NSP, _D), lambda i: (0, 0))],
        out_specs=pl.BlockSpec((_BN, _D), lambda i: (i, 0)),
        out_shape=jax.ShapeDtypeStruct((_N, _D), jnp.float32),
        interpret=_INTERP,
    )(species3, table)


def _make_spmm():
    mesh = plsc.VectorSubcoreMesh(core_axis_name="c", subcore_axis_name="s",
                                  num_cores=_NC, num_subcores=_NS)

    @functools.partial(
        pl.kernel,
        out_type=jax.ShapeDtypeStruct((_NC, _N, _D), jnp.float32),
        mesh=mesh,
        scratch_types=[
            pltpu.VMEM((_K,), jnp.int32),
            pltpu.VMEM((_K,), jnp.int32),
            pltpu.VMEM((_K,), jnp.float32),
            pltpu.VMEM((_K, _D), jnp.float32),
            pltpu.VMEM_SHARED((_N, _D), jnp.float32),
            pltpu.SemaphoreType.DMA,
        ],
    )
    def spmm(x_hbm, snd_hbm, rcv_hbm, c_hbm, out_hbm,
             sidx, ridx, cv, rows, acc, sem):
        cid = lax.axis_index("c")
        sid = lax.axis_index("s")
        wid = sid * _NC + cid
        zv = jnp.zeros((16,), jnp.float32)

        def zrow(r, carry):
            for d in range(8):
                rows[r, pl.ds(d * 16, 16)] = zv
            return carry
        lax.fori_loop(0, _K, zrow, 0)
        for j in range(-(-_NRCH // _NS)):
            rchunk = j * _NS + sid

            @pl.when(rchunk < _NRCH)
            def _():
                pltpu.sync_copy(rows, acc.at[pl.ds(rchunk * _RCH, _RCH)])
        plsc.subcore_barrier()

        base = wid * _EPW

        def chunk(g, carry):
            off = base + g * _K
            pltpu.sync_copy(snd_hbm.at[pl.ds(off, _K)], sidx)
            pltpu.sync_copy(rcv_hbm.at[pl.ds(off, _K)], ridx)
            pltpu.sync_copy(c_hbm.at[pl.ds(off, _K)], cv)
            pltpu.async_copy(x_hbm.at[sidx], rows, sem).wait()

            def scale(gi, c2):
                cg = cv[pl.ds(gi * 16, 16)]
                for j in range(16):
                    e = gi * 16 + j
                    ce = cg[j]
                    for d in range(8):
                        sl = pl.ds(d * 16, 16)
                        rows[e, sl] = rows[e, sl] * ce
                return c2
            lax.fori_loop(0, _K // 16, scale, 0)
            pltpu.sync_copy(rows, acc.at[ridx], add=True)
            return carry
        lax.fori_loop(0, _NCH, chunk, 0)

        plsc.subcore_barrier()
        for j in range(-(-_NRCH // _NS)):
            rchunk = j * _NS + sid

            @pl.when(rchunk < _NRCH)
            def _():
                r0 = rchunk * _RCH
                pltpu.sync_copy(acc.at[pl.ds(r0, _RCH)],
                                out_hbm.at[cid, pl.ds(r0, _RCH)])

    return spmm


_spmm_cache = []


def _spmm(x, snd, rcv, c):
    if not _spmm_cache:
        _spmm_cache.append(_make_spmm())
    return _spmm_cache[0](x, snd, rcv, c)


def _node_update_body(parts_ref, x_ref, sp_ref, wsc_ref, wspec_ref, lns_ref,
                      out_ref):
    agg = parts_ref[0] + parts_ref[1]
    s = sp_ref[0, 0, :]
    oh = (s[:, None] == lax.broadcasted_iota(jnp.int32, (_BN, _NSP), 1)
          ).astype(jnp.float32)
    y = (jnp.dot(agg, wsc_ref[...], preferred_element_type=jnp.float32)
         + jnp.dot(oh, wspec_ref[...], preferred_element_type=jnp.float32))
    mu = jnp.mean(y, axis=-1, keepdims=True)
    dlt = y - mu
    var = jnp.mean(dlt * dlt, axis=-1, keepdims=True)
    out_ref[...] = dlt * lax.rsqrt(var + 1e-6) * lns_ref[...] + x_ref[...]


def _node_update(parts, x, species3, Wsc, Wspec, lns2):
    return pl.pallas_call(
        _node_update_body,
        grid=(_NBLK,),
        in_specs=[pl.BlockSpec((_NC, _BN, _D), lambda i: (0, i, 0)),
                  pl.BlockSpec((_BN, _D), lambda i: (i, 0)),
                  pl.BlockSpec((1, 1, _BN), lambda i: (i, 0, 0)),
                  pl.BlockSpec((_D, _D), lambda i: (0, 0)),
                  pl.BlockSpec((_NSP, _D), lambda i: (0, 0)),
                  pl.BlockSpec((1, _D), lambda i: (0, 0))],
        out_specs=pl.BlockSpec((_BN, _D), lambda i: (i, 0)),
        out_shape=jax.ShapeDtypeStruct((_N, _D), jnp.float32),
        interpret=_INTERP,
    )(parts, x, species3, Wsc, Wspec, lns2)


def _readout_body(x_ref, wro_ref, hw1_ref, hw2_ref, gid_ref, out_ref):
    i = pl.program_id(0)
    ro = jnp.sum(x_ref[...] * wro_ref[...], axis=-1)
    t = ro[:, None] * hw1_ref[...]
    pn = jnp.sum(_silu(t) * hw2_ref[...], axis=-1)
    g = gid_ref[0, 0, :]
    oh_t = (lax.broadcasted_iota(jnp.int32, (_G, _BN), 0) == g[None, :]
            ).astype(jnp.float32)
    contrib = jnp.dot(oh_t, jnp.broadcast_to(pn[:, None], (_BN, _D)),
                      preferred_element_type=jnp.float32)

    @pl.when(i == 0)
    def _():
        out_ref[...] = jnp.zeros_like(out_ref)
    out_ref[...] += contrib


def _readout(x, wro2, hW1, hw2r, gid3):
    return pl.pallas_call(
        _readout_body,
        grid=(_NBLK,),
        in_specs=[pl.BlockSpec((_BN, _D), lambda i: (i, 0)),
                  pl.BlockSpec((1, _D), lambda i: (0, 0)),
                  pl.BlockSpec((1, 16), lambda i: (0, 0)),
                  pl.BlockSpec((1, 16), lambda i: (0, 0)),
                  pl.BlockSpec((1, 1, _BN), lambda i: (i, 0, 0))],
        out_specs=pl.BlockSpec((_G, _D), lambda i: (0, 0)),
        out_shape=jax.ShapeDtypeStruct((_G, _D), jnp.float32),
        interpret=_INTERP,
    )(x, wro2, hW1, hw2r, gid3)


def kernel(vectors, node_species, senders, receivers, graph_id, species_table,
           W1_0, W2_0, Wsc_0, Wspec_0, lns_0,
           W1_1, W2_1, Wsc_1, Wspec_1, lns_1,
           W1_2, W2_2, Wsc_2, Wspec_2, lns_2,
           W_ro, hW1, hW2):
    species3 = node_species.astype(jnp.int32).reshape(_NBLK, 1, _BN)
    gid3 = graph_id.astype(jnp.int32).reshape(_NBLK, 1, _BN)
    snd = senders.astype(jnp.int32)
    rcv = receivers.astype(jnp.int32)

    c0, c1, c2 = _edge_coeffs(vectors, W1_0, W2_0, W1_1, W2_1, W1_2, W2_2)
    cs = (c0.reshape(_E), c1.reshape(_E), c2.reshape(_E))

    x = _embed(species3, species_table)
    layer_params = ((Wsc_0, Wspec_0, lns_0), (Wsc_1, Wspec_1, lns_1),
                    (Wsc_2, Wspec_2, lns_2))
    for (Wsc, Wspec, lns), cl in zip(layer_params, cs):
        parts = _spmm(x, snd, rcv, cl)
        x = _node_update(parts, x, species3, Wsc, Wspec, lns.reshape(1, _D))

    out = _readout(x, W_ro.reshape(1, _D), hW1, hW2.reshape(1, 16), gid3)
    return out[:, 0:1]


# speedup vs baseline: 6.7783x; 3.4106x over previous
import functools

_INTERP = False

import jax
import jax.numpy as jnp
from jax import lax
from jax.experimental import pallas as pl
from jax.experimental.pallas import tpu as pltpu
from jax.experimental.pallas import tpu_sc as plsc

_N = 10000
_E = 320000
_D = 128
_NB = 8
_HB = 32
_NSH = 9
_NSP = 89
_G = 64
_RMAX = 5.0
_AVG = 32.0

_BE = 6400
_BN = 400
_NBLK = _N // _BN

_NC = 2
_NS = 16
_NW = _NC * _NS
_EPW = _E // _NW
_K = 80
_NCH = _EPW // _K
_RCH = 80
_NRCH = _N // _RCH


def _silu(t):
    return t * (1.0 / (1.0 + jnp.exp(-t)))


def _edge_coeff_body(vec_ref, w10, w20, w11, w21, w12, w22,
                     c0_ref, c1_ref, c2_ref):
    v = -vec_ref[...]
    x2 = jnp.sum(v * v, axis=0, keepdims=True)
    r = jnp.where(x2 == 0.0, 0.0,
                  jnp.sqrt(jnp.where(x2 == 0.0, 1.0, x2)))
    inv = 1.0 / jnp.where(r == 0.0, 1.0, r)
    n = v * inv
    nx, ny, nz = n[0:1], n[1:2], n[2:3]
    s3 = 3.0 ** 0.5
    s15 = 15.0 ** 0.5
    sh = jnp.concatenate([
        jnp.ones_like(nx),
        s3 * nx, s3 * ny, s3 * nz,
        s15 * nx * ny,
        s15 * ny * nz,
        ((5.0 ** 0.5) / 2.0) * (3.0 * nz * nz - 1.0),
        s15 * nx * nz,
        (s15 / 2.0) * (nx * nx - ny * ny),
    ], axis=0)

    rs = jnp.where(r == 0.0, 1.0, r)
    nvals = (lax.broadcasted_iota(jnp.int32, (_NB, 1), 0)
             .astype(jnp.float32) + 1.0)
    basis = ((2.0 / _RMAX) ** 0.5) * jnp.sin(nvals * (jnp.pi / _RMAX) * r) / rs
    u = jnp.clip(r * (1.0 / _RMAX), 0.0, 1.0)
    u3 = u * u * u
    env = 1.0 - 10.0 * u3 + 15.0 * u3 * u - 6.0 * u3 * u * u
    rad = basis * env

    for wa, wb, c_ref in ((w10, w20, c0_ref), (w11, w21, c1_ref),
                          (w12, w22, c2_ref)):
        h = _silu(jnp.dot(wa[...], rad, preferred_element_type=jnp.float32))
        w = jnp.dot(wb[...], h, preferred_element_type=jnp.float32)
        c = jnp.sum(w * sh, axis=0, keepdims=True)
        c_ref[...] = c * (1.0 / _AVG)


def _edge_coeffs(vecT, W1T_0, W2T_0, W1T_1, W2T_1, W1T_2, W2T_2):
    wspec = pl.BlockSpec((_HB, _NB), lambda i: (0, 0))
    w2spec = pl.BlockSpec((_NSH, _HB), lambda i: (0, 0))
    cspec = pl.BlockSpec((1, _BE), lambda i: (0, i))
    return pl.pallas_call(
        _edge_coeff_body,
        grid=(_E // _BE,),
        in_specs=[pl.BlockSpec((3, _BE), lambda i: (0, i)),
                  wspec, w2spec, wspec, w2spec, wspec, w2spec],
        out_specs=[cspec, cspec, cspec],
        out_shape=[jax.ShapeDtypeStruct((1, _E), jnp.float32)] * 3,
        interpret=_INTERP,
    )(vecT, W1T_0, W2T_0, W1T_1, W2T_1, W1T_2, W2T_2)


def _embed_body(sp_ref, table_ref, out_ref):
    s = sp_ref[0, 0, :]
    oh = (s[:, None] == lax.broadcasted_iota(jnp.int32, (_BN, _NSP), 1)
          ).astype(jnp.float32)
    out_ref[...] = jnp.dot(oh, table_ref[...],
                           preferred_element_type=jnp.float32)


def _embed(species3, table):
    return pl.pallas_call(
        _embed_body,
        grid=(_NBLK,),
        in_specs=[pl.BlockSpec((1, 1, _BN), lambda i: (i, 0, 0)),
                  pl.BlockSpec((_NSP, _D), lambda i: (0, 0))],
        out_specs=pl.BlockSpec((_BN, _D), lambda i: (i, 0)),
        out_shape=jax.ShapeDtypeStruct((_N, _D), jnp.float32),
        interpret=_INTERP,
    )(species3, table)


def _make_spmm():
    mesh = plsc.VectorSubcoreMesh(core_axis_name="c", subcore_axis_name="s",
                                  num_cores=_NC, num_subcores=_NS)

    @functools.partial(
        pl.kernel,
        out_type=jax.ShapeDtypeStruct((_NC, _N, _D), jnp.float32),
        mesh=mesh,
        scratch_types=[
            pltpu.VMEM((_K,), jnp.int32),
            pltpu.VMEM((_K,), jnp.int32),
            pltpu.VMEM((_K,), jnp.int32),
            pltpu.VMEM((_K,), jnp.int32),
            pltpu.VMEM((_K,), jnp.float32),
            pltpu.VMEM((_K,), jnp.float32),
            pltpu.VMEM((_K, _D), jnp.float32),
            pltpu.VMEM((_K, _D), jnp.float32),
            pltpu.VMEM_SHARED((_N, _D), jnp.float32),
            pltpu.SemaphoreType.DMA,
            pltpu.SemaphoreType.DMA,
            pltpu.SemaphoreType.DMA,
            pltpu.SemaphoreType.DMA,
        ],
    )
    def spmm(x_hbm, snd_hbm, rcv_hbm, c_hbm, out_hbm,
             sidx0, sidx1, ridx0, ridx1, cv0, cv1, rows0, rows1, acc,
             semm0, semm1, semg0, semg1):
        cid = lax.axis_index("c")
        sid = lax.axis_index("s")
        wid = sid * _NC + cid
        zv = jnp.zeros((16,), jnp.float32)
        sidx = (sidx0, sidx1)
        ridx = (ridx0, ridx1)
        cv = (cv0, cv1)
        rows = (rows0, rows1)
        semm = (semm0, semm1)
        semg = (semg0, semg1)

        def meta_issue(g, p):
            pltpu.async_copy(snd_hbm.at[wid, g], sidx[p], semm[p])
            pltpu.async_copy(rcv_hbm.at[wid, g], ridx[p], semm[p])
            pltpu.async_copy(c_hbm.at[wid, g], cv[p], semm[p])

        def meta_wait(p):
            pltpu.make_async_copy(snd_hbm.at[wid, 0], sidx[p], semm[p]).wait()
            pltpu.make_async_copy(rcv_hbm.at[wid, 0], ridx[p], semm[p]).wait()
            pltpu.make_async_copy(c_hbm.at[wid, 0], cv[p], semm[p]).wait()

        def gather_issue(p):
            pltpu.async_copy(x_hbm.at[sidx[p]], rows[p], semg[p])

        def gather_wait(p):
            pltpu.make_async_copy(x_hbm.at[sidx[p]], rows[p], semg[p]).wait()

        def scale(p):
            buf = rows[p]
            for gi in range(_K // 16):
                cg = cv[p][pl.ds(gi * 16, 16)]
                for j in range(16):
                    e = gi * 16 + j
                    ce = cg[j]
                    for d in range(8):
                        sl = pl.ds(d * 16, 16)
                        buf[e, sl] = buf[e, sl] * ce

        def scatter(p):
            pltpu.sync_copy(rows[p], acc.at[ridx[p]], add=True)

        meta_issue(0, 0)
        meta_wait(0)
        gather_issue(0)
        meta_issue(1, 1)

        def zrow(r, carry):
            for d in range(8):
                rows1[r, pl.ds(d * 16, 16)] = zv
            return carry
        lax.fori_loop(0, _K, zrow, 0)
        for j in range(-(-_NRCH // _NS)):
            rchunk = j * _NS + sid

            @pl.when(rchunk < _NRCH)
            def _():
                pltpu.sync_copy(rows1, acc.at[pl.ds(rchunk * _RCH, _RCH)])
        plsc.subcore_barrier()

        def pair(i, carry):
            g0 = 2 * i
            meta_wait(1)
            gather_issue(1)
            gather_wait(0)
            scale(0)
            scatter(0)
            meta_issue(g0 + 2, 0)
            gather_wait(1)
            scale(1)
            scatter(1)
            meta_wait(0)
            gather_issue(0)

            @pl.when(g0 + 3 < _NCH)
            def _():
                meta_issue(g0 + 3, 1)
            return carry
        lax.fori_loop(0, (_NCH - 1) // 2, pair, 0)
        gather_wait(0)
        scale(0)
        scatter(0)

        plsc.subcore_barrier()
        for j in range(-(-_NRCH // _NS)):
            rchunk = j * _NS + sid

            @pl.when(rchunk < _NRCH)
            def _():
                r0 = rchunk * _RCH
                pltpu.sync_copy(acc.at[pl.ds(r0, _RCH)],
                                out_hbm.at[cid, pl.ds(r0, _RCH)])

    return spmm


_spmm_cache = []


def _spmm(x, snd, rcv, c):
    if not _spmm_cache:
        _spmm_cache.append(_make_spmm())
    return _spmm_cache[0](x, snd, rcv, c)


def _node_update_body(parts_ref, x_ref, sp_ref, wsc_ref, wspec_ref, lns_ref,
                      out_ref):
    agg = parts_ref[0] + parts_ref[1]
    s = sp_ref[0, 0, :]
    oh = (s[:, None] == lax.broadcasted_iota(jnp.int32, (_BN, _NSP), 1)
          ).astype(jnp.float32)
    y = (jnp.dot(agg, wsc_ref[...], preferred_element_type=jnp.float32)
         + jnp.dot(oh, wspec_ref[...], preferred_element_type=jnp.float32))
    mu = jnp.mean(y, axis=-1, keepdims=True)
    dlt = y - mu
    var = jnp.mean(dlt * dlt, axis=-1, keepdims=True)
    out_ref[...] = dlt * lax.rsqrt(var + 1e-6) * lns_ref[...] + x_ref[...]


def _node_update(parts, x, species3, Wsc, Wspec, lns2):
    return pl.pallas_call(
        _node_update_body,
        grid=(_NBLK,),
        in_specs=[pl.BlockSpec((_NC, _BN, _D), lambda i: (0, i, 0)),
                  pl.BlockSpec((_BN, _D), lambda i: (i, 0)),
                  pl.BlockSpec((1, 1, _BN), lambda i: (i, 0, 0)),
                  pl.BlockSpec((_D, _D), lambda i: (0, 0)),
                  pl.BlockSpec((_NSP, _D), lambda i: (0, 0)),
                  pl.BlockSpec((1, _D), lambda i: (0, 0))],
        out_specs=pl.BlockSpec((_BN, _D), lambda i: (i, 0)),
        out_shape=jax.ShapeDtypeStruct((_N, _D), jnp.float32),
        interpret=_INTERP,
    )(parts, x, species3, Wsc, Wspec, lns2)


def _readout_body(x_ref, wro_ref, hw1_ref, hw2_ref, gid_ref, out_ref):
    i = pl.program_id(0)
    ro = jnp.sum(x_ref[...] * wro_ref[...], axis=-1)
    t = ro[:, None] * hw1_ref[...]
    pn = jnp.sum(_silu(t) * hw2_ref[...], axis=-1)
    g = gid_ref[0, 0, :]
    oh_t = (lax.broadcasted_iota(jnp.int32, (_G, _BN), 0) == g[None, :]
            ).astype(jnp.float32)
    contrib = jnp.dot(oh_t, jnp.broadcast_to(pn[:, None], (_BN, _D)),
                      preferred_element_type=jnp.float32)

    @pl.when(i == 0)
    def _():
        out_ref[...] = jnp.zeros_like(out_ref)
    out_ref[...] += contrib


def _readout(x, wro2, hW1, hw2r, gid3):
    return pl.pallas_call(
        _readout_body,
        grid=(_NBLK,),
        in_specs=[pl.BlockSpec((_BN, _D), lambda i: (i, 0)),
                  pl.BlockSpec((1, _D), lambda i: (0, 0)),
                  pl.BlockSpec((1, 16), lambda i: (0, 0)),
                  pl.BlockSpec((1, 16), lambda i: (0, 0)),
                  pl.BlockSpec((1, 1, _BN), lambda i: (i, 0, 0))],
        out_specs=pl.BlockSpec((_G, _D), lambda i: (0, 0)),
        out_shape=jax.ShapeDtypeStruct((_G, _D), jnp.float32),
        interpret=_INTERP,
    )(x, wro2, hW1, hw2r, gid3)


def kernel(vectors, node_species, senders, receivers, graph_id, species_table,
           W1_0, W2_0, Wsc_0, Wspec_0, lns_0,
           W1_1, W2_1, Wsc_1, Wspec_1, lns_1,
           W1_2, W2_2, Wsc_2, Wspec_2, lns_2,
           W_ro, hW1, hW2):
    species3 = node_species.astype(jnp.int32).reshape(_NBLK, 1, _BN)
    gid3 = graph_id.astype(jnp.int32).reshape(_NBLK, 1, _BN)
    snd = senders.astype(jnp.int32).reshape(_NW, _NCH, _K)
    rcv = receivers.astype(jnp.int32).reshape(_NW, _NCH, _K)

    c0, c1, c2 = _edge_coeffs(vectors.T, W1_0.T, W2_0.T, W1_1.T, W2_1.T,
                              W1_2.T, W2_2.T)
    cs = tuple(c.reshape(_NW, _NCH, _K) for c in (c0, c1, c2))

    x = _embed(species3, species_table)
    layer_params = ((Wsc_0, Wspec_0, lns_0), (Wsc_1, Wspec_1, lns_1),
                    (Wsc_2, Wspec_2, lns_2))
    for (Wsc, Wspec, lns), cl in zip(layer_params, cs):
        parts = _spmm(x, snd, rcv, cl)
        x = _node_update(parts, x, species3, Wsc, Wspec, lns.reshape(1, _D))

    out = _readout(x, W_ro.reshape(1, _D), hW1, hW2.reshape(1, 16), gid3)
    return out[:, 0:1]


# speedup vs baseline: 7.4396x; 1.0976x over previous
import functools

import jax
import jax.numpy as jnp
from jax import lax
from jax.experimental import pallas as pl
from jax.experimental.pallas import tpu as pltpu
from jax.experimental.pallas import tpu_sc as plsc

_N = 10000
_E = 320000
_D = 128
_NB = 8
_HB = 32
_NSH = 9
_NSP = 89
_G = 64
_RMAX = 5.0
_AVG = 32.0

_BE = 6400
_BN = 400
_NBLK = _N // _BN

_NC = 2
_NS = 16
_NW = _NC * _NS
_EPW = _E // _NW
_K = 80
_NCH = _EPW // _K
_RCH = 80
_NRCH = _N // _RCH


def _silu(t):
    return t * (1.0 / (1.0 + jnp.exp(-t)))


def _edge_coeff_body(vec_ref, *refs):
    v = -vec_ref[...]
    x2 = jnp.sum(v * v, axis=0, keepdims=True)
    r = jnp.where(x2 == 0.0, 0.0,
                  jnp.sqrt(jnp.where(x2 == 0.0, 1.0, x2)))
    inv = 1.0 / jnp.where(r == 0.0, 1.0, r)
    n = v * inv
    nx, ny, nz = n[0:1], n[1:2], n[2:3]
    s3 = 3.0 ** 0.5
    s15 = 15.0 ** 0.5
    sh = jnp.concatenate([
        jnp.ones_like(nx),
        s3 * nx, s3 * ny, s3 * nz,
        s15 * nx * ny,
        s15 * ny * nz,
        ((5.0 ** 0.5) / 2.0) * (3.0 * nz * nz - 1.0),
        s15 * nx * nz,
        (s15 / 2.0) * (nx * nx - ny * ny),
    ], axis=0)

    rs = jnp.where(r == 0.0, 1.0, r)
    theta = (jnp.pi / _RMAX) * r
    s1 = jnp.sin(theta)
    c1 = jnp.cos(theta)
    sins = [s1, 2.0 * c1 * s1]
    for _ in range(_NB - 2):
        sins.append(2.0 * c1 * sins[-1] - sins[-2])
    basis = ((2.0 / _RMAX) ** 0.5) * jnp.concatenate(sins, axis=0) / rs
    u = jnp.clip(r * (1.0 / _RMAX), 0.0, 1.0)
    u3 = u * u * u
    env = 1.0 - 10.0 * u3 + 15.0 * u3 * u - 6.0 * u3 * u * u
    rad = basis * env

    nlayers = len(refs) // 3
    for li in range(nlayers):
        wa, wb, c_ref = refs[2 * li], refs[2 * li + 1], refs[2 * nlayers + li]
        h = _silu(jnp.dot(wa[...], rad, preferred_element_type=jnp.float32,
                 precision=lax.Precision.HIGHEST))
        w = jnp.dot(wb[...], h, preferred_element_type=jnp.float32,
                 precision=lax.Precision.HIGHEST)
        c = jnp.sum(w * sh, axis=0, keepdims=True)
        c_ref[...] = c * (1.0 / _AVG)


def _edge_coeffs(vecT, wpairs):
    wspec = pl.BlockSpec((_HB, _NB), lambda i: (0, 0))
    w2spec = pl.BlockSpec((_NSH, _HB), lambda i: (0, 0))
    cspec = pl.BlockSpec((1, _BE), lambda i: (0, i))
    nl = len(wpairs)
    args = [w for pair in wpairs for w in pair]
    return pl.pallas_call(
        _edge_coeff_body,
        grid=(_E // _BE,),
        in_specs=[pl.BlockSpec((3, _BE), lambda i: (0, i))]
                 + [wspec, w2spec] * nl,
        out_specs=[cspec] * nl,
        out_shape=[jax.ShapeDtypeStruct((1, _E), jnp.float32)] * nl,
    )(vecT, *args)


def _embed_body(sp_ref, table_ref, out_ref):
    s = sp_ref[0, 0, :]
    oh = (s[:, None] == lax.broadcasted_iota(jnp.int32, (_BN, _NSP), 1)
          ).astype(jnp.float32)
    out_ref[...] = jnp.dot(oh, table_ref[...],
                           preferred_element_type=jnp.float32,
                 precision=lax.Precision.HIGHEST)


def _embed(species3, table):
    return pl.pallas_call(
        _embed_body,
        grid=(_NBLK,),
        in_specs=[pl.BlockSpec((1, 1, _BN), lambda i: (i, 0, 0)),
                  pl.BlockSpec((_NSP, _D), lambda i: (0, 0))],
        out_specs=pl.BlockSpec((_BN, _D), lambda i: (i, 0)),
        out_shape=jax.ShapeDtypeStruct((_N, _D), jnp.float32),
    )(species3, table)


def _make_spmm():
    mesh = plsc.VectorSubcoreMesh(core_axis_name="c", subcore_axis_name="s",
                                  num_cores=_NC, num_subcores=_NS)

    @functools.partial(
        pl.kernel,
        out_type=jax.ShapeDtypeStruct((_NC, _N, _D), jnp.float32),
        mesh=mesh,
        scratch_types=[
            pltpu.VMEM((_K,), jnp.int32),
            pltpu.VMEM((_K,), jnp.int32),
            pltpu.VMEM((_K,), jnp.int32),
            pltpu.VMEM((_K,), jnp.int32),
            pltpu.VMEM((_K,), jnp.float32),
            pltpu.VMEM((_K,), jnp.float32),
            pltpu.VMEM((_K, _D), jnp.float32),
            pltpu.VMEM((_K, _D), jnp.float32),
            pltpu.VMEM_SHARED((_N, _D), jnp.float32),
            pltpu.SemaphoreType.DMA,
            pltpu.SemaphoreType.DMA,
            pltpu.SemaphoreType.DMA,
            pltpu.SemaphoreType.DMA,
            pltpu.SemaphoreType.DMA,
            pltpu.SemaphoreType.DMA,
        ],
    )
    def spmm(x_hbm, snd_hbm, rcv_hbm, c_hbm, out_hbm,
             sidx0, sidx1, ridx0, ridx1, cv0, cv1, rows0, rows1, acc,
             semm0, semm1, semg0, semg1, sems0, sems1):
        cid = lax.axis_index("c")
        sid = lax.axis_index("s")
        wid = sid * _NC + cid
        zv = jnp.zeros((16,), jnp.float32)
        sidx = (sidx0, sidx1)
        ridx = (ridx0, ridx1)
        cv = (cv0, cv1)
        rows = (rows0, rows1)
        semm = (semm0, semm1)
        semg = (semg0, semg1)
        sems = (sems0, sems1)
        base = wid * _EPW

        def meta_issue(g, p):
            off = base + g * _K
            pltpu.async_copy(snd_hbm.at[pl.ds(off, _K)], sidx[p], semm[p])
            pltpu.async_copy(rcv_hbm.at[pl.ds(off, _K)], ridx[p], semm[p])
            pltpu.async_copy(c_hbm.at[pl.ds(off, _K)], cv[p], semm[p])

        def meta_wait(p):
            pltpu.make_async_copy(snd_hbm.at[pl.ds(0, _K)], sidx[p],
                                  semm[p]).wait()
            pltpu.make_async_copy(rcv_hbm.at[pl.ds(0, _K)], ridx[p],
                                  semm[p]).wait()
            pltpu.make_async_copy(c_hbm.at[pl.ds(0, _K)], cv[p],
                                  semm[p]).wait()

        def gather_issue(p):
            pltpu.async_copy(x_hbm.at[sidx[p]], rows[p], semg[p])

        def gather_wait(p):
            pltpu.make_async_copy(x_hbm.at[sidx[p]], rows[p], semg[p]).wait()

        def scale(p):
            buf = rows[p]
            cvp = cv[p]

            def grp(gi, carry):
                cg = cvp[pl.ds(gi * 16, 16)]
                for j in range(16):
                    e = gi * 16 + j
                    ce = cg[j]
                    for d in range(8):
                        sl = pl.ds(d * 16, 16)
                        buf[e, sl] = buf[e, sl] * ce
                return carry
            lax.fori_loop(0, _K // 16, grp, 0)

        def scatter_issue(p):
            pltpu.async_copy(rows[p], acc.at[ridx[p]], sems[p], add=True)

        def scatter_wait(p):
            pltpu.make_async_copy(rows[p], acc.at[ridx[p]], sems[p]).wait()

        meta_issue(0, 0)
        meta_wait(0)
        gather_issue(0)
        meta_issue(1, 1)

        def zrow(r, carry):
            for d in range(8):
                rows1[r, pl.ds(d * 16, 16)] = zv
            return carry
        lax.fori_loop(0, _K, zrow, 0)
        for j in range(-(-_NRCH // _NS)):
            rchunk = j * _NS + sid

            @pl.when(rchunk < _NRCH)
            def _():
                pltpu.sync_copy(rows1, acc.at[pl.ds(rchunk * _RCH, _RCH)])
        plsc.subcore_barrier()

        def pair(i, carry):
            g0 = 2 * i
            meta_wait(1)

            @pl.when(i > 0)
            def _():
                scatter_wait(1)
            gather_issue(1)
            gather_wait(0)
            scale(0)
            scatter_issue(0)
            meta_issue(g0 + 2, 0)
            gather_wait(1)
            scale(1)
            scatter_issue(1)
            meta_wait(0)
            scatter_wait(0)
            gather_issue(0)

            @pl.when(g0 + 3 < _NCH)
            def _():
                meta_issue(g0 + 3, 1)
            return carry
        lax.fori_loop(0, (_NCH - 1) // 2, pair, 0)
        scatter_wait(1)
        gather_wait(0)
        scale(0)
        pltpu.sync_copy(rows0, acc.at[ridx0], add=True)

        plsc.subcore_barrier()
        for j in range(-(-_NRCH // _NS)):
            rchunk = j * _NS + sid

            @pl.when(rchunk < _NRCH)
            def _():
                r0 = rchunk * _RCH
                pltpu.sync_copy(acc.at[pl.ds(r0, _RCH)],
                                out_hbm.at[cid, pl.ds(r0, _RCH)])

    return spmm


_spmm_cache = []


def _spmm(x, snd, rcv, c):
    if not _spmm_cache:
        _spmm_cache.append(_make_spmm())
    return _spmm_cache[0](x, snd, rcv, c)


def _node_update_body(parts_ref, x_ref, sp_ref, wsc_ref, wspec_ref, lns_ref,
                      out_ref):
    agg = parts_ref[0] + parts_ref[1]
    s = sp_ref[0, 0, :]
    oh = (s[:, None] == lax.broadcasted_iota(jnp.int32, (_BN, _NSP), 1)
          ).astype(jnp.float32)
    y = (jnp.dot(agg, wsc_ref[...], preferred_element_type=jnp.float32,
                 precision=lax.Precision.HIGHEST)
         + jnp.dot(oh, wspec_ref[...], preferred_element_type=jnp.float32,
                 precision=lax.Precision.HIGHEST))
    mu = jnp.mean(y, axis=-1, keepdims=True)
    dlt = y - mu
    var = jnp.mean(dlt * dlt, axis=-1, keepdims=True)
    out_ref[...] = dlt * lax.rsqrt(var + 1e-6) * lns_ref[...] + x_ref[...]


def _node_update(parts, x, species3, Wsc, Wspec, lns2):
    return pl.pallas_call(
        _node_update_body,
        grid=(_NBLK,),
        in_specs=[pl.BlockSpec((_NC, _BN, _D), lambda i: (0, i, 0)),
                  pl.BlockSpec((_BN, _D), lambda i: (i, 0)),
                  pl.BlockSpec((1, 1, _BN), lambda i: (i, 0, 0)),
                  pl.BlockSpec((_D, _D), lambda i: (0, 0)),
                  pl.BlockSpec((_NSP, _D), lambda i: (0, 0)),
                  pl.BlockSpec((1, _D), lambda i: (0, 0))],
        out_specs=pl.BlockSpec((_BN, _D), lambda i: (i, 0)),
        out_shape=jax.ShapeDtypeStruct((_N, _D), jnp.float32),
    )(parts, x, species3, Wsc, Wspec, lns2)


def _readout_body(x_ref, wro_ref, hw1_ref, hw2_ref, gid_ref, out_ref):
    i = pl.program_id(0)
    ro = jnp.sum(x_ref[...] * wro_ref[...], axis=-1)
    t = ro[:, None] * hw1_ref[...]
    pn = jnp.sum(_silu(t) * hw2_ref[...], axis=-1)
    g = gid_ref[0, 0, :]
    oh_t = (lax.broadcasted_iota(jnp.int32, (_G, _BN), 0) == g[None, :]
            ).astype(jnp.float32)
    contrib = jnp.dot(oh_t, jnp.broadcast_to(pn[:, None], (_BN, _D)),
                      preferred_element_type=jnp.float32,
                 precision=lax.Precision.HIGHEST)

    @pl.when(i == 0)
    def _():
        out_ref[...] = jnp.zeros_like(out_ref)
    out_ref[...] += contrib


def _readout(x, wro2, hW1, hw2r, gid3):
    return pl.pallas_call(
        _readout_body,
        grid=(_NBLK,),
        in_specs=[pl.BlockSpec((_BN, _D), lambda i: (i, 0)),
                  pl.BlockSpec((1, _D), lambda i: (0, 0)),
                  pl.BlockSpec((1, 16), lambda i: (0, 0)),
                  pl.BlockSpec((1, 16), lambda i: (0, 0)),
                  pl.BlockSpec((1, 1, _BN), lambda i: (i, 0, 0))],
        out_specs=pl.BlockSpec((_G, _D), lambda i: (0, 0)),
        out_shape=jax.ShapeDtypeStruct((_G, _D), jnp.float32),
    )(x, wro2, hW1, hw2r, gid3)


def kernel(vectors, node_species, senders, receivers, graph_id, species_table,
           W1_0, W2_0, Wsc_0, Wspec_0, lns_0,
           W1_1, W2_1, Wsc_1, Wspec_1, lns_1,
           W1_2, W2_2, Wsc_2, Wspec_2, lns_2,
           W_ro, hW1, hW2):
    species3 = node_species.astype(jnp.int32).reshape(_NBLK, 1, _BN)
    gid3 = graph_id.astype(jnp.int32).reshape(_NBLK, 1, _BN)
    snd = senders.astype(jnp.int32)
    rcv = receivers.astype(jnp.int32)

    vecT = vectors.T
    (c0,) = _edge_coeffs(vecT, [(W1_0.T, W2_0.T)])
    x = _embed(species3, species_table)
    parts = _spmm(x, snd, rcv, c0.reshape(_E))
    c1, c2 = _edge_coeffs(vecT, [(W1_1.T, W2_1.T), (W1_2.T, W2_2.T)])
    layer_params = ((Wsc_0, Wspec_0, lns_0), (Wsc_1, Wspec_1, lns_1),
                    (Wsc_2, Wspec_2, lns_2))
    for li, (Wsc, Wspec, lns) in enumerate(layer_params):
        x = _node_update(parts, x, species3, Wsc, Wspec, lns.reshape(1, _D))
        if li < 2:
            parts = _spmm(x, snd, rcv, (c1, c2)[li].reshape(_E))

    out = _readout(x, W_ro.reshape(1, _D), hW1, hW2.reshape(1, 16), gid3)
    return out[:, 0:1]
